# baseline (speedup 1.0000x reference)
import jax
import jax.numpy as jnp
from jax import lax
from jax.experimental import pallas as pl
from jax.experimental.pallas import tpu as pltpu

_N = 8192
_D = 512
_NSEG = 8
_BM = 4096
_NB = _N // _BM
_NEG = -1e30


def _fused_kernel(seg_ref, x_ref, w_ref, b_ref, o_ref,
                  cache_ref, st_ref, m_ref, d_ref, a_ref, b2_ref, c2_ref):
    p = pl.program_id(0)
    i = pl.program_id(1)

    @pl.when((p == 0) & (i == 0))
    def _init():
        m_ref[...] = jnp.full_like(m_ref, _NEG)
        d_ref[...] = jnp.zeros_like(d_ref)
        a_ref[...] = jnp.zeros_like(a_ref)
        b2_ref[...] = jnp.zeros_like(b2_ref)
        c2_ref[...] = jnp.zeros_like(c2_ref)

    @pl.when(p == 0)
    def _stats():
        x = x_ref[...]
        cache_ref[pl.ds(i * _BM, _BM), :] = x
        y = jnp.dot(x, w_ref[...],
                    preferred_element_type=jnp.float32) + b_ref[...]
        y_t = y.T
        lw = y_t[0:1, :]
        gw = y_t[1:2, :]
        ls = jax.nn.sigmoid(lw)
        seg = seg_ref[0]
        oh = lax.broadcasted_iota(jnp.int32, (_NSEG, _BM), 0) == seg
        gmask = jnp.where(oh, gw, _NEG)
        bmax = jnp.max(gmask, axis=1, keepdims=True)
        m_old = m_ref[...]
        m_new = jnp.maximum(m_old, bmax)
        alpha = jnp.exp(m_old - m_new)
        e = jnp.exp(gmask - m_new) * oh.astype(jnp.float32)
        v = ls * e
        d_ref[...] = d_ref[...] * alpha + jnp.sum(e, axis=1, keepdims=True)
        a_ref[...] = a_ref[...] * alpha + jnp.sum(v, axis=1, keepdims=True)
        b2_ref[...] = b2_ref[...] * alpha + jnp.dot(
            v, x, preferred_element_type=jnp.float32)
        c2_ref[...] = c2_ref[...] * alpha + jnp.dot(
            v, x * x, preferred_element_type=jnp.float32)
        m_ref[...] = m_new

        @pl.when(i == _NB - 1)
        def _fin():
            dd = d_ref[...]
            inv_d = jnp.where(dd > 0, 1.0 / dd, 0.0)
            s0 = jnp.sum(a_ref[...] * inv_d, keepdims=True)
            s1 = jnp.sum(b2_ref[...] * inv_d, axis=0, keepdims=True)
            s2 = jnp.sum(c2_ref[...] * inv_d, axis=0, keepdims=True)
            mean = s1 / s0
            var = s2 / s0 - mean * mean
            st_ref[0:1, :] = mean
            st_ref[1:2, :] = lax.rsqrt(var)

    @pl.when(p == 1)
    def _norm():
        x = cache_ref[pl.ds(i * _BM, _BM), :]
        o_ref[...] = (x - st_ref[0:1, :]) * st_ref[1:2, :]


def kernel(feats, segment_ids, local_W, local_b, global_W, global_b):
    w_cat = jnp.concatenate([local_W, global_W], axis=1)
    b_cat = jnp.concatenate([local_b, global_b]).reshape(1, 2)
    seg3 = segment_ids.reshape(_NB, 1, _BM)

    out = pl.pallas_call(
        _fused_kernel,
        grid=(2, _NB),
        in_specs=[
            pl.BlockSpec((1, 1, _BM), lambda p, i: (i * (1 - p), 0, 0)),
            pl.BlockSpec((_BM, _D), lambda p, i: (i * (1 - p), 0)),
            pl.BlockSpec((_D, 2), lambda p, i: (0, 0)),
            pl.BlockSpec((1, 2), lambda p, i: (0, 0)),
        ],
        out_specs=pl.BlockSpec((_BM, _D), lambda p, i: (i * p, 0)),
        out_shape=jax.ShapeDtypeStruct((_N, _D), jnp.float32),
        scratch_shapes=[
            pltpu.VMEM((_N, _D), jnp.float32),
            pltpu.VMEM((2, _D), jnp.float32),
            pltpu.VMEM((_NSEG, 1), jnp.float32),
            pltpu.VMEM((_NSEG, 1), jnp.float32),
            pltpu.VMEM((_NSEG, 1), jnp.float32),
            pltpu.VMEM((_NSEG, _D), jnp.float32),
            pltpu.VMEM((_NSEG, _D), jnp.float32),
        ],
        compiler_params=pltpu.CompilerParams(
            dimension_semantics=("arbitrary", "arbitrary")),
    )(seg3, feats, w_cat, b_cat)
    return out


# speedup vs baseline: 1.0393x; 1.0393x over previous
import jax
import jax.numpy as jnp
from jax import lax
from jax.experimental import pallas as pl
from jax.experimental.pallas import tpu as pltpu

_N = 8192
_D = 512
_NSEG = 8
_BM = 2048
_NB = _N // _BM
_NEG = -1e30


def _fused_kernel(seg_ref, x_ref, w_ref, b_ref, o_ref,
                  cache_ref, st_ref, m_ref, d_ref, a_ref, b2_ref, c2_ref):
    p = pl.program_id(0)
    i = pl.program_id(1)

    @pl.when((p == 0) & (i == 0))
    def _init():
        m_ref[...] = jnp.full_like(m_ref, _NEG)
        d_ref[...] = jnp.zeros_like(d_ref)
        a_ref[...] = jnp.zeros_like(a_ref)
        b2_ref[...] = jnp.zeros_like(b2_ref)
        c2_ref[...] = jnp.zeros_like(c2_ref)

    @pl.when(p == 0)
    def _stats():
        x = x_ref[...]
        cache_ref[pl.ds(i * _BM, _BM), :] = x
        y = jnp.dot(x, w_ref[...],
                    preferred_element_type=jnp.float32) + b_ref[...]
        y_t = y.T
        lw = y_t[0:1, :]
        gw = y_t[1:2, :]
        ls = jax.nn.sigmoid(lw)
        seg = seg_ref[0]
        oh = lax.broadcasted_iota(jnp.int32, (_NSEG, _BM), 0) == seg
        gmask = jnp.where(oh, gw, _NEG)
        bmax = jnp.max(gmask, axis=1, keepdims=True)
        m_old = m_ref[...]
        m_new = jnp.maximum(m_old, bmax)
        alpha = jnp.exp(m_old - m_new)
        e = jnp.exp(gmask - m_new) * oh.astype(jnp.float32)
        v = ls * e
        d_ref[...] = d_ref[...] * alpha + jnp.sum(e, axis=1, keepdims=True)
        a_ref[...] = a_ref[...] * alpha + jnp.sum(v, axis=1, keepdims=True)
        b2_ref[...] = b2_ref[...] * alpha + jnp.dot(
            v, x, preferred_element_type=jnp.float32)
        c2_ref[...] = c2_ref[...] * alpha + jnp.dot(
            v, x * x, preferred_element_type=jnp.float32)
        m_ref[...] = m_new

        @pl.when(i == _NB - 1)
        def _fin():
            dd = d_ref[...]
            inv_d = jnp.where(dd > 0, 1.0 / dd, 0.0)
            s0 = jnp.sum(a_ref[...] * inv_d, keepdims=True)
            s1 = jnp.sum(b2_ref[...] * inv_d, axis=0, keepdims=True)
            s2 = jnp.sum(c2_ref[...] * inv_d, axis=0, keepdims=True)
            mean = s1 / s0
            var = s2 / s0 - mean * mean
            rstd = lax.rsqrt(var)
            st_ref[0:1, :] = rstd
            st_ref[1:2, :] = -mean * rstd

    @pl.when(p == 1)
    def _norm():
        x = cache_ref[pl.ds(i * _BM, _BM), :]
        o_ref[...] = x * st_ref[0:1, :] + st_ref[1:2, :]


def kernel(feats, segment_ids, local_W, local_b, global_W, global_b):
    w_cat = jnp.concatenate([local_W, global_W], axis=1)
    b_cat = jnp.concatenate([local_b, global_b]).reshape(1, 2)
    seg3 = segment_ids.reshape(_NB, 1, _BM)

    out = pl.pallas_call(
        _fused_kernel,
        grid=(2, _NB),
        in_specs=[
            pl.BlockSpec((1, 1, _BM), lambda p, i: (i * (1 - p), 0, 0)),
            pl.BlockSpec((_BM, _D), lambda p, i: (i * (1 - p), 0)),
            pl.BlockSpec((_D, 2), lambda p, i: (0, 0)),
            pl.BlockSpec((1, 2), lambda p, i: (0, 0)),
        ],
        out_specs=pl.BlockSpec((_BM, _D), lambda p, i: (i * p, 0)),
        out_shape=jax.ShapeDtypeStruct((_N, _D), jnp.float32),
        scratch_shapes=[
            pltpu.VMEM((_N, _D), jnp.float32),
            pltpu.VMEM((2, _D), jnp.float32),
            pltpu.VMEM((_NSEG, 1), jnp.float32),
            pltpu.VMEM((_NSEG, 1), jnp.float32),
            pltpu.VMEM((_NSEG, 1), jnp.float32),
            pltpu.VMEM((_NSEG, _D), jnp.float32),
            pltpu.VMEM((_NSEG, _D), jnp.float32),
        ],
        compiler_params=pltpu.CompilerParams(
            dimension_semantics=("arbitrary", "arbitrary")),
    )(seg3, feats, w_cat, b_cat)
    return out


# speedup vs baseline: 1.1395x; 1.0964x over previous
import jax
import jax.numpy as jnp
from jax import lax
from jax.experimental import pallas as pl
from jax.experimental.pallas import tpu as pltpu

_N = 8192
_D = 512
_NSEG = 8
_BM = 2048
_NB = _N // _BM
_NEG = -1e30


def _fused_kernel(seg_ref, x_ref, w_ref, b_ref, o_ref,
                  cache_ref, st_ref, m_ref, d_ref, a_ref, b2_ref, c2_ref,
                  sem_ref):
    p = pl.program_id(0)
    i = pl.program_id(1)

    @pl.when((p == 0) & (i == 0))
    def _init():
        m_ref[...] = jnp.full_like(m_ref, _NEG)
        d_ref[...] = jnp.zeros_like(d_ref)
        a_ref[...] = jnp.zeros_like(a_ref)
        b2_ref[...] = jnp.zeros_like(b2_ref)
        c2_ref[...] = jnp.zeros_like(c2_ref)

    @pl.when(p == 0)
    def _stats():
        copy = pltpu.make_async_copy(
            x_ref, cache_ref.at[pl.ds(i * _BM, _BM), :], sem_ref)
        copy.start()
        x = x_ref[...]
        xb = x.astype(jnp.bfloat16)
        y_t = lax.dot_general(
            w_ref[...], xb, (((0,), (1,)), ((), ())),
            preferred_element_type=jnp.float32) + b_ref[...]
        lw = y_t[0:1, :]
        gw = y_t[1:2, :]
        ls = jax.nn.sigmoid(lw)
        seg = seg_ref[0]
        oh = lax.broadcasted_iota(jnp.int32, (_NSEG, _BM), 0) == seg
        gmask = jnp.where(oh, gw, _NEG)
        bmax = jnp.max(gmask, axis=1, keepdims=True)
        m_old = m_ref[...]
        m_new = jnp.maximum(m_old, bmax)
        alpha = jnp.exp(m_old - m_new)
        e = jnp.exp(gmask - m_new) * oh.astype(jnp.float32)
        v = ls * e
        d_ref[...] = d_ref[...] * alpha + jnp.sum(e, axis=1, keepdims=True)
        a_ref[...] = a_ref[...] * alpha + jnp.sum(v, axis=1, keepdims=True)
        vb = v.astype(jnp.bfloat16)
        b2_ref[...] = b2_ref[...] * alpha + jnp.dot(
            vb, xb, preferred_element_type=jnp.float32)
        c2_ref[...] = c2_ref[...] * alpha + jnp.dot(
            vb, xb * xb, preferred_element_type=jnp.float32)
        m_ref[...] = m_new
        copy.wait()

        @pl.when(i == _NB - 1)
        def _fin():
            dd = d_ref[...]
            inv_d = jnp.where(dd > 0, 1.0 / dd, 0.0)
            s0 = jnp.sum(a_ref[...] * inv_d, keepdims=True)
            s1 = jnp.sum(b2_ref[...] * inv_d, axis=0, keepdims=True)
            s2 = jnp.sum(c2_ref[...] * inv_d, axis=0, keepdims=True)
            mean = s1 / s0
            var = s2 / s0 - mean * mean
            rstd = lax.rsqrt(var)
            st_ref[0:1, :] = rstd
            st_ref[1:2, :] = -mean * rstd

    @pl.when(p == 1)
    def _norm():
        x = cache_ref[pl.ds(i * _BM, _BM), :]
        o_ref[...] = x * st_ref[0:1, :] + st_ref[1:2, :]


def kernel(feats, segment_ids, local_W, local_b, global_W, global_b):
    w_cat = jnp.concatenate([local_W, global_W], axis=1).astype(jnp.bfloat16)
    b_cat = jnp.concatenate([local_b, global_b]).reshape(2, 1)
    seg3 = segment_ids.reshape(_NB, 1, _BM)

    out = pl.pallas_call(
        _fused_kernel,
        grid=(2, _NB),
        in_specs=[
            pl.BlockSpec((1, 1, _BM), lambda p, i: (i * (1 - p), 0, 0)),
            pl.BlockSpec((_BM, _D), lambda p, i: (i * (1 - p), 0)),
            pl.BlockSpec((_D, 2), lambda p, i: (0, 0)),
            pl.BlockSpec((2, 1), lambda p, i: (0, 0)),
        ],
        out_specs=pl.BlockSpec((_BM, _D), lambda p, i: (i * p, 0)),
        out_shape=jax.ShapeDtypeStruct((_N, _D), jnp.float32),
        scratch_shapes=[
            pltpu.VMEM((_N, _D), jnp.float32),
            pltpu.VMEM((2, _D), jnp.float32),
            pltpu.VMEM((_NSEG, 1), jnp.float32),
            pltpu.VMEM((_NSEG, 1), jnp.float32),
            pltpu.VMEM((_NSEG, 1), jnp.float32),
            pltpu.VMEM((_NSEG, _D), jnp.float32),
            pltpu.VMEM((_NSEG, _D), jnp.float32),
            pltpu.SemaphoreType.DMA,
        ],
        compiler_params=pltpu.CompilerParams(
            dimension_semantics=("arbitrary", "arbitrary")),
    )(seg3, feats, w_cat, b_cat)
    return out


# speedup vs baseline: 1.1503x; 1.0095x over previous
import jax
import jax.numpy as jnp
from jax import lax
from jax.experimental import pallas as pl
from jax.experimental.pallas import tpu as pltpu

_N = 8192
_D = 512
_NSEG = 8
_BM = 2048
_NB = _N // _BM
_NEG = -1e30


def _fused_kernel(seg_ref, x_hbm, w_ref, b_ref, o_ref,
                  cache_ref, st_ref, m_ref, d_ref, a_ref, b2_ref, c2_ref,
                  sem_ref):
    p = pl.program_id(0)
    i = pl.program_id(1)

    def _block_copy(j):
        return pltpu.make_async_copy(
            x_hbm.at[pl.ds(j * _BM, _BM), :],
            cache_ref.at[pl.ds(j * _BM, _BM), :],
            sem_ref.at[j % 2])

    @pl.when((p == 0) & (i == 0))
    def _init():
        m_ref[...] = jnp.full_like(m_ref, _NEG)
        d_ref[...] = jnp.zeros_like(d_ref)
        a_ref[...] = jnp.zeros_like(a_ref)
        b2_ref[...] = jnp.zeros_like(b2_ref)
        c2_ref[...] = jnp.zeros_like(c2_ref)
        _block_copy(0).start()

    @pl.when(p == 0)
    def _stats():
        @pl.when(i + 1 < _NB)
        def _prefetch():
            _block_copy(i + 1).start()

        _block_copy(i).wait()
        x = cache_ref[pl.ds(i * _BM, _BM), :]
        xb = x.astype(jnp.bfloat16)
        y_t = lax.dot_general(
            w_ref[...], xb, (((0,), (1,)), ((), ())),
            preferred_element_type=jnp.float32) + b_ref[...]
        lw = y_t[0:1, :]
        gw = y_t[1:2, :]
        ls = jax.nn.sigmoid(lw)
        seg = seg_ref[0]
        oh = lax.broadcasted_iota(jnp.int32, (_NSEG, _BM), 0) == seg
        gmask = jnp.where(oh, gw, _NEG)
        bmax = jnp.max(gmask, axis=1, keepdims=True)
        m_old = m_ref[...]
        m_new = jnp.maximum(m_old, bmax)
        alpha = jnp.exp(m_old - m_new)
        e = jnp.exp(gmask - m_new) * oh.astype(jnp.float32)
        v = ls * e
        d_ref[...] = d_ref[...] * alpha + jnp.sum(e, axis=1, keepdims=True)
        a_ref[...] = a_ref[...] * alpha + jnp.sum(v, axis=1, keepdims=True)
        vb = v.astype(jnp.bfloat16)
        b2_ref[...] = b2_ref[...] * alpha + jnp.dot(
            vb, xb, preferred_element_type=jnp.float32)
        c2_ref[...] = c2_ref[...] * alpha + jnp.dot(
            vb, xb * xb, preferred_element_type=jnp.float32)
        m_ref[...] = m_new

        @pl.when(i == _NB - 1)
        def _fin():
            dd = d_ref[...]
            inv_d = jnp.where(dd > 0, 1.0 / dd, 0.0)
            s0 = jnp.sum(a_ref[...] * inv_d, keepdims=True)
            s1 = jnp.sum(b2_ref[...] * inv_d, axis=0, keepdims=True)
            s2 = jnp.sum(c2_ref[...] * inv_d, axis=0, keepdims=True)
            mean = s1 / s0
            var = s2 / s0 - mean * mean
            rstd = lax.rsqrt(var)
            st_ref[0:1, :] = rstd
            st_ref[1:2, :] = -mean * rstd

    @pl.when(p == 1)
    def _norm():
        x = cache_ref[pl.ds(i * _BM, _BM), :]
        o_ref[...] = x * st_ref[0:1, :] + st_ref[1:2, :]


def kernel(feats, segment_ids, local_W, local_b, global_W, global_b):
    w_cat = jnp.concatenate([local_W, global_W], axis=1).astype(jnp.bfloat16)
    b_cat = jnp.concatenate([local_b, global_b]).reshape(2, 1)
    seg3 = segment_ids.reshape(_NB, 1, _BM)

    out = pl.pallas_call(
        _fused_kernel,
        grid=(2, _NB),
        in_specs=[
            pl.BlockSpec((1, 1, _BM), lambda p, i: (i * (1 - p), 0, 0)),
            pl.BlockSpec(memory_space=pl.ANY),
            pl.BlockSpec((_D, 2), lambda p, i: (0, 0)),
            pl.BlockSpec((2, 1), lambda p, i: (0, 0)),
        ],
        out_specs=pl.BlockSpec((_BM, _D), lambda p, i: (i * p, 0)),
        out_shape=jax.ShapeDtypeStruct((_N, _D), jnp.float32),
        scratch_shapes=[
            pltpu.VMEM((_N, _D), jnp.float32),
            pltpu.VMEM((2, _D), jnp.float32),
            pltpu.VMEM((_NSEG, 1), jnp.float32),
            pltpu.VMEM((_NSEG, 1), jnp.float32),
            pltpu.VMEM((_NSEG, 1), jnp.float32),
            pltpu.VMEM((_NSEG, _D), jnp.float32),
            pltpu.VMEM((_NSEG, _D), jnp.float32),
            pltpu.SemaphoreType.DMA((2,)),
        ],
        compiler_params=pltpu.CompilerParams(
            dimension_semantics=("arbitrary", "arbitrary")),
    )(seg3, feats, w_cat, b_cat)
    return out


# speedup vs baseline: 1.1807x; 1.0264x over previous
import jax
import jax.numpy as jnp
from jax import lax
from jax.experimental import pallas as pl
from jax.experimental.pallas import tpu as pltpu

_N = 8192
_D = 512
_NSEG = 8
_BM = 2048
_NB = _N // _BM
_NEG = -1e30


def _fused_kernel(seg_ref, x_hbm, w_ref, b_ref, o_ref,
                  cache_ref, st_ref, m_ref, d_ref, a_ref, b2_ref, c2_ref,
                  sem_ref):
    p = pl.program_id(0)
    i = pl.program_id(1)

    def _block_copy(j):
        return pltpu.make_async_copy(
            x_hbm.at[pl.ds(j * _BM, _BM), :],
            cache_ref.at[pl.ds(j * _BM, _BM), :],
            sem_ref.at[j])

    @pl.when((p == 0) & (i == 0))
    def _init():
        m_ref[...] = jnp.full_like(m_ref, _NEG)
        d_ref[...] = jnp.zeros_like(d_ref)
        a_ref[...] = jnp.zeros_like(a_ref)
        b2_ref[...] = jnp.zeros_like(b2_ref)
        c2_ref[...] = jnp.zeros_like(c2_ref)
        for j in range(_NB):
            _block_copy(j).start()

    @pl.when(p == 0)
    def _stats():
        _block_copy(i).wait()
        x = cache_ref[pl.ds(i * _BM, _BM), :]
        xb = x.astype(jnp.bfloat16)
        y_t = lax.dot_general(
            w_ref[...], xb, (((0,), (1,)), ((), ())),
            preferred_element_type=jnp.float32) + b_ref[...]
        lw = y_t[0:1, :]
        gw = y_t[1:2, :]
        ls = jax.nn.sigmoid(lw)
        seg = seg_ref[0]
        oh = lax.broadcasted_iota(jnp.int32, (_NSEG, _BM), 0) == seg
        gmask = jnp.where(oh, gw, _NEG)
        bmax = jnp.max(gmask, axis=1, keepdims=True)
        m_old = m_ref[...]
        m_new = jnp.maximum(m_old, bmax)
        alpha = jnp.exp(m_old - m_new)
        e = jnp.exp(gmask - m_new) * oh.astype(jnp.float32)
        v = ls * e
        d_ref[...] = d_ref[...] * alpha + jnp.sum(e, axis=1, keepdims=True)
        a_ref[...] = a_ref[...] * alpha + jnp.sum(v, axis=1, keepdims=True)
        vb = v.astype(jnp.bfloat16)
        b2_ref[...] = b2_ref[...] * alpha + jnp.dot(
            vb, xb, preferred_element_type=jnp.float32)
        c2_ref[...] = c2_ref[...] * alpha + jnp.dot(
            vb, xb * xb, preferred_element_type=jnp.float32)
        m_ref[...] = m_new

        @pl.when(i == _NB - 1)
        def _fin():
            dd = d_ref[...]
            inv_d = jnp.where(dd > 0, 1.0 / dd, 0.0)
            s0 = jnp.sum(a_ref[...] * inv_d, keepdims=True)
            s1 = jnp.sum(b2_ref[...] * inv_d, axis=0, keepdims=True)
            s2 = jnp.sum(c2_ref[...] * inv_d, axis=0, keepdims=True)
            mean = s1 / s0
            var = s2 / s0 - mean * mean
            rstd = lax.rsqrt(var)
            st_ref[0:1, :] = rstd
            st_ref[1:2, :] = -mean * rstd

    @pl.when(p == 1)
    def _norm():
        x = cache_ref[pl.ds(i * _BM, _BM), :]
        o_ref[...] = x * st_ref[0:1, :] + st_ref[1:2, :]


def kernel(feats, segment_ids, local_W, local_b, global_W, global_b):
    w_cat = jnp.concatenate([local_W, global_W], axis=1).astype(jnp.bfloat16)
    b_cat = jnp.concatenate([local_b, global_b]).reshape(2, 1)
    seg3 = segment_ids.reshape(_NB, 1, _BM)

    out = pl.pallas_call(
        _fused_kernel,
        grid=(2, _NB),
        in_specs=[
            pl.BlockSpec((1, 1, _BM), lambda p, i: (i * (1 - p), 0, 0)),
            pl.BlockSpec(memory_space=pl.ANY),
            pl.BlockSpec((_D, 2), lambda p, i: (0, 0)),
            pl.BlockSpec((2, 1), lambda p, i: (0, 0)),
        ],
        out_specs=pl.BlockSpec((_BM, _D), lambda p, i: (i * p, 0)),
        out_shape=jax.ShapeDtypeStruct((_N, _D), jnp.float32),
        scratch_shapes=[
            pltpu.VMEM((_N, _D), jnp.float32),
            pltpu.VMEM((2, _D), jnp.float32),
            pltpu.VMEM((_NSEG, 1), jnp.float32),
            pltpu.VMEM((_NSEG, 1), jnp.float32),
            pltpu.VMEM((_NSEG, 1), jnp.float32),
            pltpu.VMEM((_NSEG, _D), jnp.float32),
            pltpu.VMEM((_NSEG, _D), jnp.float32),
            pltpu.SemaphoreType.DMA((_NB,)),
        ],
        compiler_params=pltpu.CompilerParams(
            dimension_semantics=("arbitrary", "arbitrary")),
    )(seg3, feats, w_cat, b_cat)
    return out
